# Initial kernel scaffold; baseline (speedup 1.0000x reference)
import jax
import jax.numpy as jnp
from jax.experimental import pallas as pl


def kernel(emb, gat_W, gat_A, gat_gamma, gat_beta, W_out, b_out, gru_Wih0, gru_Whh0, gru_bih0, gru_bhh0, gru_Wih1, gru_Whh1, gru_bih1, gru_bhh1, wid, edge_index, graph_ids):
    raise NotImplementedError("write your pallas kernel here")


# speedup vs baseline: 1.2923x; 1.2923x over previous
import functools
import jax
import jax.numpy as jnp
from jax import lax
from jax.experimental import pallas as pl

N_NODES = 10000
N_GRAPHS = 500
VOCAB = 1000
H = 128
NB = 1000
GP = 512


def _prep_body(wid_ref, emb_ref, W_ref, A_ref, h0_ref, z_ref, s_ref, bmax_ref):
    i = pl.program_id(0)
    wid = wid_ref[0, 0, :]
    onehot = jnp.where(
        wid[:, None] == lax.broadcasted_iota(jnp.int32, (NB, VOCAB), 1),
        1.0, 0.0).astype(jnp.float32)
    h0 = jnp.dot(onehot, emb_ref[...], preferred_element_type=jnp.float32)
    h0_ref[...] = h0
    svecs = []
    for k in range(4):
        Wk = W_ref[k]
        zk = lax.dot_general(h0, Wk, (((1,), (1,)), ((), ())),
                             preferred_element_type=jnp.float32)
        z_ref[k] = zk
        a = A_ref[k, 0, :]
        s1 = jnp.dot(zk, a[:H], preferred_element_type=jnp.float32)
        s2 = jnp.dot(zk, a[H:], preferred_element_type=jnp.float32)
        svecs.append(s1)
        svecs.append(s2)
    sblk = jnp.stack(svecs, axis=0)
    s_ref[0] = sblk
    blkmax = jnp.max(sblk, axis=1, keepdims=True)

    @pl.when(i == 0)
    def _():
        bmax_ref[...] = jnp.full((8, 128), -1e30, jnp.float32)
    bmax_ref[...] = jnp.maximum(bmax_ref[...], jnp.broadcast_to(blkmax, (8, 128)))


def _prep(wid3, emb, gat_W, gat_A):
    grid = N_NODES // NB
    return pl.pallas_call(
        _prep_body,
        grid=(grid,),
        in_specs=[
            pl.BlockSpec((1, 1, NB), lambda i: (i, 0, 0)),
            pl.BlockSpec((VOCAB, H), lambda i: (0, 0)),
            pl.BlockSpec((4, H, H), lambda i: (0, 0, 0)),
            pl.BlockSpec((4, 1, 2 * H), lambda i: (0, 0, 0)),
        ],
        out_specs=[
            pl.BlockSpec((NB, H), lambda i: (i, 0)),
            pl.BlockSpec((4, NB, H), lambda i: (0, i, 0)),
            pl.BlockSpec((1, 8, NB), lambda i: (i, 0, 0)),
            pl.BlockSpec((8, 128), lambda i: (0, 0)),
        ],
        out_shape=[
            jax.ShapeDtypeStruct((N_NODES, H), jnp.float32),
            jax.ShapeDtypeStruct((4, N_NODES, H), jnp.float32),
            jax.ShapeDtypeStruct((N_NODES // NB, 8, NB), jnp.float32),
            jax.ShapeDtypeStruct((8, 128), jnp.float32),
        ],
    )(wid3, emb, gat_W, gat_A)


def _edges_jnp(z_all, svec, B, src, dst):
    hmsg_l, ssum_l = [], []
    for k in range(4):
        e = svec[2 * k][src] + svec[2 * k + 1][dst]
        e = jnp.maximum(e, 0.01 * e)
        w = jnp.exp(e - B[k])
        ssum = jax.ops.segment_sum(w, dst, num_segments=N_NODES)
        hm = jax.ops.segment_sum(w[:, None] * z_all[k][src], dst,
                                 num_segments=N_NODES)
        hmsg_l.append(hm)
        ssum_l.append(ssum)
    return jnp.stack(hmsg_l), jnp.stack(ssum_l)


def _stats_body(hm_ref, ss_ref, hr_ref, st_ref):
    i = pl.program_id(0)

    @pl.when(i == 0)
    def _():
        st_ref[...] = jnp.zeros((8, 128), jnp.float32)
    rows = []
    for k in range(4):
        s = ss_ref[0, k, :]
        hr = jnp.maximum(hm_ref[k] / jnp.maximum(s, 1e-16)[:, None], 0.0)
        hr_ref[k] = hr
        rows.append(jnp.sum(hr, axis=0))
        rows.append(jnp.sum(hr * hr, axis=0))
    st_ref[...] = st_ref[...] + jnp.stack(rows, axis=0)


def _stats(hmsg, ssum):
    grid = N_NODES // NB
    return pl.pallas_call(
        _stats_body,
        grid=(grid,),
        in_specs=[
            pl.BlockSpec((4, NB, H), lambda i: (0, i, 0)),
            pl.BlockSpec((1, 4, NB), lambda i: (i, 0, 0)),
        ],
        out_specs=[
            pl.BlockSpec((4, NB, H), lambda i: (0, i, 0)),
            pl.BlockSpec((8, 128), lambda i: (0, 0)),
        ],
        out_shape=[
            jax.ShapeDtypeStruct((4, N_NODES, H), jnp.float32),
            jax.ShapeDtypeStruct((8, 128), jnp.float32),
        ],
    )(hmsg, ssum)


def _final_body(hr_ref, st_ref, g_ref, b_ref, Wo_ref, bo_ref, out_ref):
    st = st_ref[...]
    acc = jnp.broadcast_to(bo_ref[0, :], (NB, H))
    for k in range(4):
        mu = st[2 * k] / float(N_NODES)
        var = st[2 * k + 1] / float(N_NODES) - mu * mu
        inv = lax.rsqrt(var + 1e-5)
        hb = (hr_ref[k] - mu[None, :]) * (inv * g_ref[k])[None, :] + b_ref[k][None, :]
        Wk = Wo_ref[:, k * H:(k + 1) * H]
        acc = acc + lax.dot_general(hb, Wk, (((1,), (1,)), ((), ())),
                                    preferred_element_type=jnp.float32)
    out_ref[...] = acc


def _finalize(hr, st, gamma, beta, W_out, b_out2):
    grid = N_NODES // NB
    return pl.pallas_call(
        _final_body,
        grid=(grid,),
        in_specs=[
            pl.BlockSpec((4, NB, H), lambda i: (0, i, 0)),
            pl.BlockSpec((8, 128), lambda i: (0, 0)),
            pl.BlockSpec((4, H), lambda i: (0, 0)),
            pl.BlockSpec((4, H), lambda i: (0, 0)),
            pl.BlockSpec((H, 4 * H), lambda i: (0, 0)),
            pl.BlockSpec((8, H), lambda i: (0, 0)),
        ],
        out_specs=pl.BlockSpec((NB, H), lambda i: (i, 0)),
        out_shape=jax.ShapeDtypeStruct((N_NODES, H), jnp.float32),
    )(hr, st, gamma, beta, W_out, b_out2)


def _segmean_body(gid_ref, h0_ref, nh_ref, sum_ref, cnt_ref):
    i = pl.program_id(0)

    @pl.when(i == 0)
    def _():
        sum_ref[...] = jnp.zeros((2, GP, H), jnp.float32)
        cnt_ref[...] = jnp.zeros((8, GP), jnp.float32)
    gid = gid_ref[0, 0, :]
    onehot = jnp.where(
        gid[:, None] == lax.broadcasted_iota(jnp.int32, (NB, GP), 1),
        1.0, 0.0).astype(jnp.float32)
    sum_ref[0] += lax.dot_general(onehot, h0_ref[...], (((0,), (0,)), ((), ())),
                                  preferred_element_type=jnp.float32)
    sum_ref[1] += lax.dot_general(onehot, nh_ref[...], (((0,), (0,)), ((), ())),
                                  preferred_element_type=jnp.float32)
    cnt = jnp.sum(onehot, axis=0)
    cnt_ref[...] += jnp.broadcast_to(cnt[None, :], (8, GP))


def _segmean(gid3, h0, new_h):
    grid = N_NODES // NB
    return pl.pallas_call(
        _segmean_body,
        grid=(grid,),
        in_specs=[
            pl.BlockSpec((1, 1, NB), lambda i: (i, 0, 0)),
            pl.BlockSpec((NB, H), lambda i: (i, 0)),
            pl.BlockSpec((NB, H), lambda i: (i, 0)),
        ],
        out_specs=[
            pl.BlockSpec((2, GP, H), lambda i: (0, 0, 0)),
            pl.BlockSpec((8, GP), lambda i: (0, 0)),
        ],
        out_shape=[
            jax.ShapeDtypeStruct((2, GP, H), jnp.float32),
            jax.ShapeDtypeStruct((8, GP), jnp.float32),
        ],
    )(gid3, h0, new_h)


def _gru_cell(x, h, Wih, Whh, bih, bhh):
    gi = lax.dot_general(x, Wih, (((1,), (1,)), ((), ())),
                         preferred_element_type=jnp.float32) + bih[None, :]
    gh = lax.dot_general(h, Whh, (((1,), (1,)), ((), ())),
                         preferred_element_type=jnp.float32) + bhh[None, :]
    ir, iz, inn = gi[:, :H], gi[:, H:2 * H], gi[:, 2 * H:]
    hr, hz, hn = gh[:, :H], gh[:, H:2 * H], gh[:, 2 * H:]
    r = jax.nn.sigmoid(ir + hr)
    z = jax.nn.sigmoid(iz + hz)
    n = jnp.tanh(inn + r * hn)
    return (1.0 - z) * n + z * h


def _gru_body(sum_ref, cnt_ref, Wih0_ref, Whh0_ref, bih0_ref, bhh0_ref,
              Wih1_ref, Whh1_ref, bih1_ref, bhh1_ref, out_ref):
    cnt = jnp.maximum(cnt_ref[0, :], 1.0)[:, None]
    x0 = sum_ref[0] / cnt
    x1 = sum_ref[1] / cnt
    zero = jnp.zeros((GP, H), jnp.float32)
    f0 = _gru_cell(x0, zero, Wih0_ref[0], Whh0_ref[0], bih0_ref[0, :], bhh0_ref[0, :])
    f1 = _gru_cell(x1, f0, Wih0_ref[0], Whh0_ref[0], bih0_ref[0, :], bhh0_ref[0, :])
    b1 = _gru_cell(x1, zero, Wih0_ref[1], Whh0_ref[1], bih0_ref[1, :], bhh0_ref[1, :])
    b0 = _gru_cell(x0, b1, Wih0_ref[1], Whh0_ref[1], bih0_ref[1, :], bhh0_ref[1, :])
    y0 = jnp.concatenate([f0, b0], axis=1)
    y1 = jnp.concatenate([f1, b1], axis=1)
    g0 = _gru_cell(y0, zero, Wih1_ref[0], Whh1_ref[0], bih1_ref[0, :], bhh1_ref[0, :])
    g1 = _gru_cell(y1, g0, Wih1_ref[0], Whh1_ref[0], bih1_ref[0, :], bhh1_ref[0, :])
    c1 = _gru_cell(y1, zero, Wih1_ref[1], Whh1_ref[1], bih1_ref[1, :], bhh1_ref[1, :])
    c0 = _gru_cell(y0, c1, Wih1_ref[1], Whh1_ref[1], bih1_ref[1, :], bhh1_ref[1, :])
    out_ref[...] = f1 + b0 + g1 + c0


def _gru(sums, cnts, Wih0, Whh0, bih0, bhh0, Wih1, Whh1, bih1, bhh1):
    full = lambda shape: pl.BlockSpec(shape, lambda: tuple(0 for _ in shape))
    return pl.pallas_call(
        _gru_body,
        in_specs=[
            full((2, GP, H)), full((8, GP)),
            full((2, 3 * H, H)), full((2, 3 * H, H)),
            full((2, 3 * H)), full((2, 3 * H)),
            full((2, 3 * H, 2 * H)), full((2, 3 * H, H)),
            full((2, 3 * H)), full((2, 3 * H)),
        ],
        out_specs=full((GP, H)),
        out_shape=jax.ShapeDtypeStruct((GP, H), jnp.float32),
    )(sums, cnts, Wih0, Whh0, bih0, bhh0, Wih1, Whh1, bih1, bhh1)


def kernel(emb, gat_W, gat_A, gat_gamma, gat_beta, W_out, b_out,
           gru_Wih0, gru_Whh0, gru_bih0, gru_bhh0,
           gru_Wih1, gru_Whh1, gru_bih1, gru_bhh1,
           wid, edge_index, graph_ids):
    wid3 = wid.astype(jnp.int32).reshape(N_NODES // NB, 1, NB)
    gid3 = graph_ids.astype(jnp.int32).reshape(N_NODES // NB, 1, NB)
    src = edge_index[0].astype(jnp.int32)
    dst = edge_index[1].astype(jnp.int32)

    h0, z_all, svec3, bmax = _prep(wid3, emb, gat_W, gat_A)
    svec = svec3.transpose(1, 0, 2).reshape(8, N_NODES)
    mx = bmax[:, 0]
    e_ub = mx[0::2] + mx[1::2]
    B = jnp.maximum(e_ub, 0.01 * e_ub)

    hmsg, ssum = _edges_jnp(z_all, svec, B, src, dst)
    ssum3 = ssum.reshape(4, N_NODES // NB, NB).transpose(1, 0, 2)

    hr, st = _stats(hmsg, ssum3)
    b_out2 = jnp.broadcast_to(b_out[None, :], (8, H))
    new_h = _finalize(hr, st, gat_gamma, gat_beta, W_out, b_out2)

    sums, cnts = _segmean(gid3, h0, new_h)
    hsum = _gru(sums, cnts, gru_Wih0, gru_Whh0, gru_bih0, gru_bhh0,
                gru_Wih1, gru_Whh1, gru_bih1, gru_bhh1)
    return new_h, hsum[:N_GRAPHS]


# speedup vs baseline: 7.9990x; 6.1895x over previous
import functools
import jax
import jax.numpy as jnp
from jax import lax
from jax.experimental import pallas as pl
from jax.experimental.pallas import tpu as pltpu
from jax.experimental.pallas import tpu_sc as plsc

N_NODES = 10000
N_GRAPHS = 500
VOCAB = 1000
H = 128
NB = 1000
GP = 512

N_EDGES = 320000
NWORK = 16
EPW = 20480
KB = 128
NBATCH = EPW // KB
NPAD = 10240
ROWS_T = NPAD // 16
NBS = 1024


def _prep_body(wid_ref, emb_ref, W_ref, A_ref, h0_ref, z_ref, s_ref, bmax_ref):
    i = pl.program_id(0)
    wid = wid_ref[0, 0, :]
    onehot = jnp.where(
        wid[:, None] == lax.broadcasted_iota(jnp.int32, (NB, VOCAB), 1),
        1.0, 0.0).astype(jnp.float32)
    h0 = jnp.dot(onehot, emb_ref[...], preferred_element_type=jnp.float32)
    h0_ref[...] = h0
    svecs = []
    for k in range(4):
        Wk = W_ref[k]
        zk = lax.dot_general(h0, Wk, (((1,), (1,)), ((), ())),
                             preferred_element_type=jnp.float32)
        z_ref[k] = zk
        a = A_ref[k, 0, :]
        s1 = jnp.dot(zk, a[:H], preferred_element_type=jnp.float32)
        s2 = jnp.dot(zk, a[H:], preferred_element_type=jnp.float32)
        svecs.append(s1)
        svecs.append(s2)
    sblk = jnp.stack(svecs, axis=0)
    s_ref[0] = sblk
    blkmax = jnp.max(sblk, axis=1, keepdims=True)

    @pl.when(i == 0)
    def _():
        bmax_ref[...] = jnp.full((8, 128), -1e30, jnp.float32)
    bmax_ref[...] = jnp.maximum(bmax_ref[...], jnp.broadcast_to(blkmax, (8, 128)))


def _prep(wid3, emb, gat_W, gat_A):
    grid = N_NODES // NB
    return pl.pallas_call(
        _prep_body,
        grid=(grid,),
        in_specs=[
            pl.BlockSpec((1, 1, NB), lambda i: (i, 0, 0)),
            pl.BlockSpec((VOCAB, H), lambda i: (0, 0)),
            pl.BlockSpec((4, H, H), lambda i: (0, 0, 0)),
            pl.BlockSpec((4, 1, 2 * H), lambda i: (0, 0, 0)),
        ],
        out_specs=[
            pl.BlockSpec((NB, H), lambda i: (i, 0)),
            pl.BlockSpec((4, NB, H), lambda i: (0, i, 0)),
            pl.BlockSpec((1, 8, NB), lambda i: (i, 0, 0)),
            pl.BlockSpec((8, 128), lambda i: (0, 0)),
        ],
        out_shape=[
            jax.ShapeDtypeStruct((N_NODES, H), jnp.float32),
            jax.ShapeDtypeStruct((4, N_NODES, H), jnp.float32),
            jax.ShapeDtypeStruct((N_NODES // NB, 8, NB), jnp.float32),
            jax.ShapeDtypeStruct((8, 128), jnp.float32),
        ],
    )(wid3, emb, gat_W, gat_A)


def _edges_sc_body(z_ref, s_ref, b_ref, src_ref, dst_ref, hp_ref, ss_ref,
                   sidx8, didx8, s1v, s2v, bv, wbuf, ssl, rin, hmsg):
    sid = lax.axis_index("s")
    wid = sid
    vcnt = jnp.clip(N_EDGES - wid * EPW, 0, EPW)

    pltpu.sync_copy(b_ref, bv)

    zv = jnp.zeros((16,), jnp.float32)
    lanes = lax.iota(jnp.int32, 16)

    for i in range(4):
        def _zero_rin(k, _):
            for j in range(H // 16):
                rin[k, pl.ds(j * 16, 16)] = zv
            return 0
        lax.fori_loop(0, KB, _zero_rin, 0)
        for q in range(ROWS_T // KB):
            pltpu.sync_copy(rin, hmsg.at[pl.ds(sid * ROWS_T + q * KB, KB)])

        def _zero_ssl(k, _):
            ssl[pl.ds(k * 16, 16)] = zv
            return 0
        lax.fori_loop(0, NPAD // 16, _zero_ssl, 0)
        pltpu.sync_copy(s_ref.at[2 * i], s1v)
        pltpu.sync_copy(s_ref.at[2 * i + 1], s2v)
        bsc = bv[...][i]
        plsc.subcore_barrier()

        def _sbatch(sb, _):
            pltpu.sync_copy(src_ref.at[wid].at[pl.ds(sb * 8, 8)], sidx8)
            pltpu.sync_copy(dst_ref.at[wid].at[pl.ds(sb * 8, 8)], didx8)

            def _one(u, _2):
                b = sb * 8 + u
                pltpu.sync_copy(z_ref.at[i].at[sidx8.at[u]], rin)
                for j in range(KB // 16):
                    s16 = sidx8[u, pl.ds(j * 16, 16)]
                    d16 = didx8[u, pl.ds(j * 16, 16)]
                    g1 = plsc.load_gather(s1v, [s16])
                    g2 = plsc.load_gather(s2v, [d16])
                    e = g1 + g2
                    e = jnp.maximum(e, 0.01 * e)
                    w = jnp.exp(e - bsc)
                    eidx = b * KB + j * 16 + lanes
                    w = jnp.where(eidx < vcnt, w, 0.0)
                    wbuf[pl.ds(j * 16, 16)] = w
                    plsc.addupdate_scatter(ssl, [d16], w)

                def _scale(jj, _3):
                    w16 = wbuf[pl.ds(jj * 16, 16)]
                    for l in range(16):
                        s = w16[l]
                        k = jj * 16 + l
                        for j in range(H // 16):
                            rin[k, pl.ds(j * 16, 16)] = (
                                rin[k, pl.ds(j * 16, 16)] * s)
                    return 0
                lax.fori_loop(0, KB // 16, _scale, 0)
                pltpu.sync_copy(rin, hmsg.at[didx8.at[u]], add=True)
                return 0
            lax.fori_loop(0, 8, _one, 0)
            return 0
        lax.fori_loop(0, NBATCH // 8, _sbatch, 0)
        pltpu.sync_copy(ssl, ss_ref.at[i].at[0].at[sid])
        plsc.subcore_barrier()
        pltpu.sync_copy(hmsg.at[pl.ds(sid * ROWS_T, ROWS_T)],
                        hp_ref.at[i].at[0].at[pl.ds(sid * ROWS_T, ROWS_T)])
        plsc.subcore_barrier()


def _edges_sc(z_all, svec, b16, srcw, dstw):
    mesh = plsc.VectorSubcoreMesh(core_axis_name="c", subcore_axis_name="s",
                                  num_cores=1)
    f = pl.kernel(
        _edges_sc_body, mesh=mesh,
        compiler_params=pltpu.CompilerParams(needs_layout_passes=False),
        out_type=[
            jax.ShapeDtypeStruct((4, 1, NPAD, H), jnp.float32),
            jax.ShapeDtypeStruct((4, 1, 16, NPAD), jnp.float32),
        ],
        scratch_types=[
            pltpu.VMEM((8, KB), jnp.int32),
            pltpu.VMEM((8, KB), jnp.int32),
            pltpu.VMEM((N_NODES,), jnp.float32),
            pltpu.VMEM((N_NODES,), jnp.float32),
            pltpu.VMEM((16,), jnp.float32),
            pltpu.VMEM((KB,), jnp.float32),
            pltpu.VMEM((NPAD,), jnp.float32),
            pltpu.VMEM((KB, H), jnp.float32),
            pltpu.VMEM_SHARED((NPAD, H), jnp.float32),
        ],
    )
    return f(z_all, svec, b16, srcw, dstw)


def _stats_body(hp_ref, ss_ref, hr_ref, st_ref):
    i = pl.program_id(0)

    @pl.when(i == 0)
    def _():
        st_ref[...] = jnp.zeros((8, 128), jnp.float32)
    rows = []
    for k in range(4):
        v = hp_ref[k, 0]
        s = jnp.sum(ss_ref[k, 0], axis=0)
        hr = jnp.maximum(v / jnp.maximum(s, 1e-16)[:, None], 0.0)
        hr_ref[k] = hr
        rows.append(jnp.sum(hr, axis=0))
        rows.append(jnp.sum(hr * hr, axis=0))
    st_ref[...] = st_ref[...] + jnp.stack(rows, axis=0)


def _stats(hp, ssp):
    grid = NPAD // NBS
    return pl.pallas_call(
        _stats_body,
        grid=(grid,),
        in_specs=[
            pl.BlockSpec((4, 1, NBS, H), lambda i: (0, 0, i, 0)),
            pl.BlockSpec((4, 1, 16, NBS), lambda i: (0, 0, 0, i)),
        ],
        out_specs=[
            pl.BlockSpec((4, NBS, H), lambda i: (0, i, 0)),
            pl.BlockSpec((8, 128), lambda i: (0, 0)),
        ],
        out_shape=[
            jax.ShapeDtypeStruct((4, NPAD, H), jnp.float32),
            jax.ShapeDtypeStruct((8, 128), jnp.float32),
        ],
    )(hp, ssp)


def _final_body(hr_ref, st_ref, g_ref, b_ref, Wo_ref, bo_ref, out_ref):
    st = st_ref[...]
    acc = jnp.broadcast_to(bo_ref[0, :], (NBS, H))
    for k in range(4):
        mu = st[2 * k] / float(N_NODES)
        var = st[2 * k + 1] / float(N_NODES) - mu * mu
        inv = lax.rsqrt(var + 1e-5)
        hb = (hr_ref[k] - mu[None, :]) * (inv * g_ref[k])[None, :] + b_ref[k][None, :]
        Wk = Wo_ref[:, k * H:(k + 1) * H]
        acc = acc + lax.dot_general(hb, Wk, (((1,), (1,)), ((), ())),
                                    preferred_element_type=jnp.float32)
    out_ref[...] = acc


def _finalize(hr, st, gamma, beta, W_out, b_out2):
    grid = NPAD // NBS
    return pl.pallas_call(
        _final_body,
        grid=(grid,),
        in_specs=[
            pl.BlockSpec((4, NBS, H), lambda i: (0, i, 0)),
            pl.BlockSpec((8, 128), lambda i: (0, 0)),
            pl.BlockSpec((4, H), lambda i: (0, 0)),
            pl.BlockSpec((4, H), lambda i: (0, 0)),
            pl.BlockSpec((H, 4 * H), lambda i: (0, 0)),
            pl.BlockSpec((8, H), lambda i: (0, 0)),
        ],
        out_specs=pl.BlockSpec((NBS, H), lambda i: (i, 0)),
        out_shape=jax.ShapeDtypeStruct((NPAD, H), jnp.float32),
    )(hr, st, gamma, beta, W_out, b_out2)


def _segmean_body(gid_ref, h0_ref, nh_ref, sum_ref, cnt_ref):
    i = pl.program_id(0)

    @pl.when(i == 0)
    def _():
        sum_ref[...] = jnp.zeros((2, GP, H), jnp.float32)
        cnt_ref[...] = jnp.zeros((8, GP), jnp.float32)
    gid = gid_ref[0, 0, :]
    onehot = jnp.where(
        gid[:, None] == lax.broadcasted_iota(jnp.int32, (NB, GP), 1),
        1.0, 0.0).astype(jnp.float32)
    sum_ref[0] += lax.dot_general(onehot, h0_ref[...], (((0,), (0,)), ((), ())),
                                  preferred_element_type=jnp.float32)
    sum_ref[1] += lax.dot_general(onehot, nh_ref[...], (((0,), (0,)), ((), ())),
                                  preferred_element_type=jnp.float32)
    cnt = jnp.sum(onehot, axis=0)
    cnt_ref[...] += jnp.broadcast_to(cnt[None, :], (8, GP))


def _segmean(gid3, h0, new_h):
    grid = N_NODES // NB
    return pl.pallas_call(
        _segmean_body,
        grid=(grid,),
        in_specs=[
            pl.BlockSpec((1, 1, NB), lambda i: (i, 0, 0)),
            pl.BlockSpec((NB, H), lambda i: (i, 0)),
            pl.BlockSpec((NB, H), lambda i: (i, 0)),
        ],
        out_specs=[
            pl.BlockSpec((2, GP, H), lambda i: (0, 0, 0)),
            pl.BlockSpec((8, GP), lambda i: (0, 0)),
        ],
        out_shape=[
            jax.ShapeDtypeStruct((2, GP, H), jnp.float32),
            jax.ShapeDtypeStruct((8, GP), jnp.float32),
        ],
    )(gid3, h0, new_h)


def _gru_cell(x, h, Wih, Whh, bih, bhh):
    gi = lax.dot_general(x, Wih, (((1,), (1,)), ((), ())),
                         preferred_element_type=jnp.float32) + bih[None, :]
    gh = lax.dot_general(h, Whh, (((1,), (1,)), ((), ())),
                         preferred_element_type=jnp.float32) + bhh[None, :]
    ir, iz, inn = gi[:, :H], gi[:, H:2 * H], gi[:, 2 * H:]
    hr, hz, hn = gh[:, :H], gh[:, H:2 * H], gh[:, 2 * H:]
    r = jax.nn.sigmoid(ir + hr)
    z = jax.nn.sigmoid(iz + hz)
    n = jnp.tanh(inn + r * hn)
    return (1.0 - z) * n + z * h


def _gru_body(sum_ref, cnt_ref, Wih0_ref, Whh0_ref, bih0_ref, bhh0_ref,
              Wih1_ref, Whh1_ref, bih1_ref, bhh1_ref, out_ref):
    cnt = jnp.maximum(cnt_ref[0, :], 1.0)[:, None]
    x0 = sum_ref[0] / cnt
    x1 = sum_ref[1] / cnt
    zero = jnp.zeros((GP, H), jnp.float32)
    f0 = _gru_cell(x0, zero, Wih0_ref[0], Whh0_ref[0], bih0_ref[0, :], bhh0_ref[0, :])
    f1 = _gru_cell(x1, f0, Wih0_ref[0], Whh0_ref[0], bih0_ref[0, :], bhh0_ref[0, :])
    b1 = _gru_cell(x1, zero, Wih0_ref[1], Whh0_ref[1], bih0_ref[1, :], bhh0_ref[1, :])
    b0 = _gru_cell(x0, b1, Wih0_ref[1], Whh0_ref[1], bih0_ref[1, :], bhh0_ref[1, :])
    y0 = jnp.concatenate([f0, b0], axis=1)
    y1 = jnp.concatenate([f1, b1], axis=1)
    g0 = _gru_cell(y0, zero, Wih1_ref[0], Whh1_ref[0], bih1_ref[0, :], bhh1_ref[0, :])
    g1 = _gru_cell(y1, g0, Wih1_ref[0], Whh1_ref[0], bih1_ref[0, :], bhh1_ref[0, :])
    c1 = _gru_cell(y1, zero, Wih1_ref[1], Whh1_ref[1], bih1_ref[1, :], bhh1_ref[1, :])
    c0 = _gru_cell(y0, c1, Wih1_ref[1], Whh1_ref[1], bih1_ref[1, :], bhh1_ref[1, :])
    out_ref[...] = f1 + b0 + g1 + c0


def _gru(sums, cnts, Wih0, Whh0, bih0, bhh0, Wih1, Whh1, bih1, bhh1):
    full = lambda shape: pl.BlockSpec(shape, lambda: tuple(0 for _ in shape))
    return pl.pallas_call(
        _gru_body,
        in_specs=[
            full((2, GP, H)), full((8, GP)),
            full((2, 3 * H, H)), full((2, 3 * H, H)),
            full((2, 3 * H)), full((2, 3 * H)),
            full((2, 3 * H, 2 * H)), full((2, 3 * H, H)),
            full((2, 3 * H)), full((2, 3 * H)),
        ],
        out_specs=full((GP, H)),
        out_shape=jax.ShapeDtypeStruct((GP, H), jnp.float32),
    )(sums, cnts, Wih0, Whh0, bih0, bhh0, Wih1, Whh1, bih1, bhh1)


def kernel(emb, gat_W, gat_A, gat_gamma, gat_beta, W_out, b_out,
           gru_Wih0, gru_Whh0, gru_bih0, gru_bhh0,
           gru_Wih1, gru_Whh1, gru_bih1, gru_bhh1,
           wid, edge_index, graph_ids):
    wid3 = wid.astype(jnp.int32).reshape(N_NODES // NB, 1, NB)
    gid3 = graph_ids.astype(jnp.int32).reshape(N_NODES // NB, 1, NB)
    src = edge_index[0].astype(jnp.int32)
    dst = edge_index[1].astype(jnp.int32)

    h0, z_all, svec3, bmax = _prep(wid3, emb, gat_W, gat_A)
    svec = svec3.transpose(1, 0, 2).reshape(8, N_NODES)
    mx = bmax[:, 0]
    e_ub = mx[0::2] + mx[1::2]
    B = jnp.maximum(e_ub, 0.01 * e_ub)
    b16 = jnp.pad(B, (0, 12))

    pad = NWORK * EPW - N_EDGES
    srcw = jnp.pad(src, (0, pad)).reshape(NWORK, NBATCH, KB)
    dstw = jnp.pad(dst, (0, pad)).reshape(NWORK, NBATCH, KB)
    hp, ssp = _edges_sc(z_all, svec, b16, srcw, dstw)

    hr, st = _stats(hp, ssp)
    b_out2 = jnp.broadcast_to(b_out[None, :], (8, H))
    new_h = _finalize(hr, st, gat_gamma, gat_beta, W_out, b_out2)[:N_NODES]

    sums, cnts = _segmean(gid3, h0, new_h)
    hsum = _gru(sums, cnts, gru_Wih0, gru_Whh0, gru_bih0, gru_bhh0,
                gru_Wih1, gru_Whh1, gru_bih1, gru_bhh1)
    return new_h, hsum[:N_GRAPHS]


# speedup vs baseline: 10.4832x; 1.3106x over previous
import functools
import jax
import jax.numpy as jnp
from jax import lax
from jax.experimental import pallas as pl
from jax.experimental.pallas import tpu as pltpu
from jax.experimental.pallas import tpu_sc as plsc

N_NODES = 10000
N_GRAPHS = 500
VOCAB = 1000
H = 128
NB = 1000
GP = 512

N_EDGES = 320000
NWORK = 32
EPW = 10240
KB = 128
NBATCH = EPW // KB
NPAD = 10240
ROWS_T = NPAD // 16
NBS = 1024


def _prep_body(wid_ref, emb_ref, W_ref, A_ref, h0_ref, z_ref, s_ref, bmax_ref):
    i = pl.program_id(0)
    wid = wid_ref[0, 0, :]
    onehot = jnp.where(
        wid[:, None] == lax.broadcasted_iota(jnp.int32, (NB, VOCAB), 1),
        1.0, 0.0).astype(jnp.float32)
    h0 = jnp.dot(onehot, emb_ref[...], preferred_element_type=jnp.float32)
    h0_ref[...] = h0
    svecs = []
    for k in range(4):
        Wk = W_ref[k]
        zk = lax.dot_general(h0, Wk, (((1,), (1,)), ((), ())),
                             preferred_element_type=jnp.float32)
        z_ref[k] = zk
        a = A_ref[k, 0, :]
        s1 = jnp.dot(zk, a[:H], preferred_element_type=jnp.float32)
        s2 = jnp.dot(zk, a[H:], preferred_element_type=jnp.float32)
        svecs.append(s1)
        svecs.append(s2)
    sblk = jnp.stack(svecs, axis=0)
    s_ref[0] = sblk
    blkmax = jnp.max(sblk, axis=1, keepdims=True)

    @pl.when(i == 0)
    def _():
        bmax_ref[...] = jnp.full((8, 128), -1e30, jnp.float32)
    bmax_ref[...] = jnp.maximum(bmax_ref[...], jnp.broadcast_to(blkmax, (8, 128)))


def _prep(wid3, emb, gat_W, gat_A):
    grid = N_NODES // NB
    return pl.pallas_call(
        _prep_body,
        grid=(grid,),
        in_specs=[
            pl.BlockSpec((1, 1, NB), lambda i: (i, 0, 0)),
            pl.BlockSpec((VOCAB, H), lambda i: (0, 0)),
            pl.BlockSpec((4, H, H), lambda i: (0, 0, 0)),
            pl.BlockSpec((4, 1, 2 * H), lambda i: (0, 0, 0)),
        ],
        out_specs=[
            pl.BlockSpec((NB, H), lambda i: (i, 0)),
            pl.BlockSpec((4, NB, H), lambda i: (0, i, 0)),
            pl.BlockSpec((1, 8, NB), lambda i: (i, 0, 0)),
            pl.BlockSpec((8, 128), lambda i: (0, 0)),
        ],
        out_shape=[
            jax.ShapeDtypeStruct((N_NODES, H), jnp.float32),
            jax.ShapeDtypeStruct((4, N_NODES, H), jnp.float32),
            jax.ShapeDtypeStruct((N_NODES // NB, 8, NB), jnp.float32),
            jax.ShapeDtypeStruct((8, 128), jnp.float32),
        ],
    )(wid3, emb, gat_W, gat_A)


def _edges_sc_body(z_ref, s_ref, b_ref, src_ref, dst_ref, hp_ref, ss_ref,
                   sidx8, didx8, s1v, s2v, bv, wbuf, ssl, rin, hmsg):
    cid = lax.axis_index("c")
    sid = lax.axis_index("s")
    wid = sid * 2 + cid
    vcnt = jnp.clip(N_EDGES - wid * EPW, 0, EPW)

    pltpu.sync_copy(b_ref, bv)

    zv = jnp.zeros((16,), jnp.float32)
    lanes = lax.iota(jnp.int32, 16)

    for i in range(4):
        def _zero_rin(k, _):
            for j in range(H // 16):
                rin[k, pl.ds(j * 16, 16)] = zv
            return 0
        lax.fori_loop(0, KB, _zero_rin, 0)
        for q in range(ROWS_T // KB):
            pltpu.sync_copy(rin, hmsg.at[pl.ds(sid * ROWS_T + q * KB, KB)])

        def _zero_ssl(k, _):
            ssl[pl.ds(k * 16, 16)] = zv
            return 0
        lax.fori_loop(0, NPAD // 16, _zero_ssl, 0)
        pltpu.sync_copy(s_ref.at[2 * i], s1v)
        pltpu.sync_copy(s_ref.at[2 * i + 1], s2v)
        bsc = bv[...][i]
        plsc.subcore_barrier()

        def _sbatch(sb, _):
            pltpu.sync_copy(src_ref.at[wid].at[pl.ds(sb * 8, 8)], sidx8)
            pltpu.sync_copy(dst_ref.at[wid].at[pl.ds(sb * 8, 8)], didx8)

            def _one(u, _2):
                b = sb * 8 + u
                pltpu.sync_copy(z_ref.at[i].at[sidx8.at[u]], rin)
                for j in range(KB // 16):
                    s16 = sidx8[u, pl.ds(j * 16, 16)]
                    d16 = didx8[u, pl.ds(j * 16, 16)]
                    g1 = plsc.load_gather(s1v, [s16])
                    g2 = plsc.load_gather(s2v, [d16])
                    e = g1 + g2
                    e = jnp.maximum(e, 0.01 * e)
                    w = jnp.exp(e - bsc)
                    eidx = b * KB + j * 16 + lanes
                    w = jnp.where(eidx < vcnt, w, 0.0)
                    wbuf[pl.ds(j * 16, 16)] = w
                    plsc.addupdate_scatter(ssl, [d16], w)

                def _scale(jj, _3):
                    w16 = wbuf[pl.ds(jj * 16, 16)]
                    for l in range(16):
                        s = w16[l]
                        k = jj * 16 + l
                        for j in range(H // 16):
                            rin[k, pl.ds(j * 16, 16)] = (
                                rin[k, pl.ds(j * 16, 16)] * s)
                    return 0
                lax.fori_loop(0, KB // 16, _scale, 0)
                pltpu.sync_copy(rin, hmsg.at[didx8.at[u]], add=True)
                return 0
            lax.fori_loop(0, 8, _one, 0)
            return 0
        lax.fori_loop(0, NBATCH // 8, _sbatch, 0)
        pltpu.sync_copy(ssl, ss_ref.at[i].at[cid].at[sid])
        plsc.subcore_barrier()
        pltpu.sync_copy(hmsg.at[pl.ds(sid * ROWS_T, ROWS_T)],
                        hp_ref.at[i].at[cid].at[pl.ds(sid * ROWS_T, ROWS_T)])
        plsc.subcore_barrier()


def _edges_sc(z_all, svec, b16, srcw, dstw):
    mesh = plsc.VectorSubcoreMesh(core_axis_name="c", subcore_axis_name="s")
    f = pl.kernel(
        _edges_sc_body, mesh=mesh,
        compiler_params=pltpu.CompilerParams(needs_layout_passes=False),
        out_type=[
            jax.ShapeDtypeStruct((4, 2, NPAD, H), jnp.float32),
            jax.ShapeDtypeStruct((4, 2, 16, NPAD), jnp.float32),
        ],
        scratch_types=[
            pltpu.VMEM((8, KB), jnp.int32),
            pltpu.VMEM((8, KB), jnp.int32),
            pltpu.VMEM((N_NODES,), jnp.float32),
            pltpu.VMEM((N_NODES,), jnp.float32),
            pltpu.VMEM((16,), jnp.float32),
            pltpu.VMEM((KB,), jnp.float32),
            pltpu.VMEM((NPAD,), jnp.float32),
            pltpu.VMEM((KB, H), jnp.float32),
            pltpu.VMEM_SHARED((NPAD, H), jnp.float32),
        ],
    )
    return f(z_all, svec, b16, srcw, dstw)


def _stats_body(hp_ref, ss_ref, hr_ref, st_ref):
    i = pl.program_id(0)

    @pl.when(i == 0)
    def _():
        st_ref[...] = jnp.zeros((8, 128), jnp.float32)
    rows = []
    for k in range(4):
        v = hp_ref[k, 0] + hp_ref[k, 1]
        s = jnp.sum(ss_ref[k], axis=(0, 1))
        hr = jnp.maximum(v / jnp.maximum(s, 1e-16)[:, None], 0.0)
        hr_ref[k] = hr
        rows.append(jnp.sum(hr, axis=0))
        rows.append(jnp.sum(hr * hr, axis=0))
    st_ref[...] = st_ref[...] + jnp.stack(rows, axis=0)


def _stats(hp, ssp):
    grid = NPAD // NBS
    return pl.pallas_call(
        _stats_body,
        grid=(grid,),
        in_specs=[
            pl.BlockSpec((4, 2, NBS, H), lambda i: (0, 0, i, 0)),
            pl.BlockSpec((4, 2, 16, NBS), lambda i: (0, 0, 0, i)),
        ],
        out_specs=[
            pl.BlockSpec((4, NBS, H), lambda i: (0, i, 0)),
            pl.BlockSpec((8, 128), lambda i: (0, 0)),
        ],
        out_shape=[
            jax.ShapeDtypeStruct((4, NPAD, H), jnp.float32),
            jax.ShapeDtypeStruct((8, 128), jnp.float32),
        ],
    )(hp, ssp)


def _final_body(hr_ref, st_ref, g_ref, b_ref, Wo_ref, bo_ref, out_ref):
    st = st_ref[...]
    acc = jnp.broadcast_to(bo_ref[0, :], (NBS, H))
    for k in range(4):
        mu = st[2 * k] / float(N_NODES)
        var = st[2 * k + 1] / float(N_NODES) - mu * mu
        inv = lax.rsqrt(var + 1e-5)
        hb = (hr_ref[k] - mu[None, :]) * (inv * g_ref[k])[None, :] + b_ref[k][None, :]
        Wk = Wo_ref[:, k * H:(k + 1) * H]
        acc = acc + lax.dot_general(hb, Wk, (((1,), (1,)), ((), ())),
                                    preferred_element_type=jnp.float32)
    out_ref[...] = acc


def _finalize(hr, st, gamma, beta, W_out, b_out2):
    grid = NPAD // NBS
    return pl.pallas_call(
        _final_body,
        grid=(grid,),
        in_specs=[
            pl.BlockSpec((4, NBS, H), lambda i: (0, i, 0)),
            pl.BlockSpec((8, 128), lambda i: (0, 0)),
            pl.BlockSpec((4, H), lambda i: (0, 0)),
            pl.BlockSpec((4, H), lambda i: (0, 0)),
            pl.BlockSpec((H, 4 * H), lambda i: (0, 0)),
            pl.BlockSpec((8, H), lambda i: (0, 0)),
        ],
        out_specs=pl.BlockSpec((NBS, H), lambda i: (i, 0)),
        out_shape=jax.ShapeDtypeStruct((NPAD, H), jnp.float32),
    )(hr, st, gamma, beta, W_out, b_out2)


def _segmean_body(gid_ref, h0_ref, nh_ref, sum_ref, cnt_ref):
    i = pl.program_id(0)

    @pl.when(i == 0)
    def _():
        sum_ref[...] = jnp.zeros((2, GP, H), jnp.float32)
        cnt_ref[...] = jnp.zeros((8, GP), jnp.float32)
    gid = gid_ref[0, 0, :]
    onehot = jnp.where(
        gid[:, None] == lax.broadcasted_iota(jnp.int32, (NB, GP), 1),
        1.0, 0.0).astype(jnp.float32)
    sum_ref[0] += lax.dot_general(onehot, h0_ref[...], (((0,), (0,)), ((), ())),
                                  preferred_element_type=jnp.float32)
    sum_ref[1] += lax.dot_general(onehot, nh_ref[...], (((0,), (0,)), ((), ())),
                                  preferred_element_type=jnp.float32)
    cnt = jnp.sum(onehot, axis=0)
    cnt_ref[...] += jnp.broadcast_to(cnt[None, :], (8, GP))


def _segmean(gid3, h0, new_h):
    grid = N_NODES // NB
    return pl.pallas_call(
        _segmean_body,
        grid=(grid,),
        in_specs=[
            pl.BlockSpec((1, 1, NB), lambda i: (i, 0, 0)),
            pl.BlockSpec((NB, H), lambda i: (i, 0)),
            pl.BlockSpec((NB, H), lambda i: (i, 0)),
        ],
        out_specs=[
            pl.BlockSpec((2, GP, H), lambda i: (0, 0, 0)),
            pl.BlockSpec((8, GP), lambda i: (0, 0)),
        ],
        out_shape=[
            jax.ShapeDtypeStruct((2, GP, H), jnp.float32),
            jax.ShapeDtypeStruct((8, GP), jnp.float32),
        ],
    )(gid3, h0, new_h)


def _gru_cell(x, h, Wih, Whh, bih, bhh):
    gi = lax.dot_general(x, Wih, (((1,), (1,)), ((), ())),
                         preferred_element_type=jnp.float32) + bih[None, :]
    gh = lax.dot_general(h, Whh, (((1,), (1,)), ((), ())),
                         preferred_element_type=jnp.float32) + bhh[None, :]
    ir, iz, inn = gi[:, :H], gi[:, H:2 * H], gi[:, 2 * H:]
    hr, hz, hn = gh[:, :H], gh[:, H:2 * H], gh[:, 2 * H:]
    r = jax.nn.sigmoid(ir + hr)
    z = jax.nn.sigmoid(iz + hz)
    n = jnp.tanh(inn + r * hn)
    return (1.0 - z) * n + z * h


def _gru_body(sum_ref, cnt_ref, Wih0_ref, Whh0_ref, bih0_ref, bhh0_ref,
              Wih1_ref, Whh1_ref, bih1_ref, bhh1_ref, out_ref):
    cnt = jnp.maximum(cnt_ref[0, :], 1.0)[:, None]
    x0 = sum_ref[0] / cnt
    x1 = sum_ref[1] / cnt
    zero = jnp.zeros((GP, H), jnp.float32)
    f0 = _gru_cell(x0, zero, Wih0_ref[0], Whh0_ref[0], bih0_ref[0, :], bhh0_ref[0, :])
    f1 = _gru_cell(x1, f0, Wih0_ref[0], Whh0_ref[0], bih0_ref[0, :], bhh0_ref[0, :])
    b1 = _gru_cell(x1, zero, Wih0_ref[1], Whh0_ref[1], bih0_ref[1, :], bhh0_ref[1, :])
    b0 = _gru_cell(x0, b1, Wih0_ref[1], Whh0_ref[1], bih0_ref[1, :], bhh0_ref[1, :])
    y0 = jnp.concatenate([f0, b0], axis=1)
    y1 = jnp.concatenate([f1, b1], axis=1)
    g0 = _gru_cell(y0, zero, Wih1_ref[0], Whh1_ref[0], bih1_ref[0, :], bhh1_ref[0, :])
    g1 = _gru_cell(y1, g0, Wih1_ref[0], Whh1_ref[0], bih1_ref[0, :], bhh1_ref[0, :])
    c1 = _gru_cell(y1, zero, Wih1_ref[1], Whh1_ref[1], bih1_ref[1, :], bhh1_ref[1, :])
    c0 = _gru_cell(y0, c1, Wih1_ref[1], Whh1_ref[1], bih1_ref[1, :], bhh1_ref[1, :])
    out_ref[...] = f1 + b0 + g1 + c0


def _gru(sums, cnts, Wih0, Whh0, bih0, bhh0, Wih1, Whh1, bih1, bhh1):
    full = lambda shape: pl.BlockSpec(shape, lambda: tuple(0 for _ in shape))
    return pl.pallas_call(
        _gru_body,
        in_specs=[
            full((2, GP, H)), full((8, GP)),
            full((2, 3 * H, H)), full((2, 3 * H, H)),
            full((2, 3 * H)), full((2, 3 * H)),
            full((2, 3 * H, 2 * H)), full((2, 3 * H, H)),
            full((2, 3 * H)), full((2, 3 * H)),
        ],
        out_specs=full((GP, H)),
        out_shape=jax.ShapeDtypeStruct((GP, H), jnp.float32),
    )(sums, cnts, Wih0, Whh0, bih0, bhh0, Wih1, Whh1, bih1, bhh1)


def kernel(emb, gat_W, gat_A, gat_gamma, gat_beta, W_out, b_out,
           gru_Wih0, gru_Whh0, gru_bih0, gru_bhh0,
           gru_Wih1, gru_Whh1, gru_bih1, gru_bhh1,
           wid, edge_index, graph_ids):
    wid3 = wid.astype(jnp.int32).reshape(N_NODES // NB, 1, NB)
    gid3 = graph_ids.astype(jnp.int32).reshape(N_NODES // NB, 1, NB)
    src = edge_index[0].astype(jnp.int32)
    dst = edge_index[1].astype(jnp.int32)

    h0, z_all, svec3, bmax = _prep(wid3, emb, gat_W, gat_A)
    svec = svec3.transpose(1, 0, 2).reshape(8, N_NODES)
    mx = bmax[:, 0]
    e_ub = mx[0::2] + mx[1::2]
    B = jnp.maximum(e_ub, 0.01 * e_ub)
    b16 = jnp.pad(B, (0, 12))

    pad = NWORK * EPW - N_EDGES
    srcw = jnp.pad(src, (0, pad)).reshape(NWORK, NBATCH, KB)
    dstw = jnp.pad(dst, (0, pad)).reshape(NWORK, NBATCH, KB)
    hp, ssp = _edges_sc(z_all, svec, b16, srcw, dstw)

    hr, st = _stats(hp, ssp)
    b_out2 = jnp.broadcast_to(b_out[None, :], (8, H))
    new_h = _finalize(hr, st, gat_gamma, gat_beta, W_out, b_out2)[:N_NODES]

    sums, cnts = _segmean(gid3, h0, new_h)
    hsum = _gru(sums, cnts, gru_Wih0, gru_Whh0, gru_bih0, gru_bhh0,
                gru_Wih1, gru_Whh1, gru_bih1, gru_bhh1)
    return new_h, hsum[:N_GRAPHS]


# speedup vs baseline: 10.9794x; 1.0473x over previous
import functools
import jax
import jax.numpy as jnp
from jax import lax
from jax.experimental import pallas as pl
from jax.experimental.pallas import tpu as pltpu
from jax.experimental.pallas import tpu_sc as plsc

N_NODES = 10000
N_GRAPHS = 500
VOCAB = 1000
H = 128
NB = 1000
GP = 512

N_EDGES = 320000
NWORK = 32
EPW = 10240
KB = 128
NBATCH = EPW // KB
SBN = NBATCH // 8
NPAD = 10240
ROWS_T = NPAD // 16
NBS = 1024


def _prep_body(wid_ref, emb_ref, W_ref, A_ref, h0_ref, z_ref, s_ref, bmax_ref):
    i = pl.program_id(0)
    wid = wid_ref[0, 0, :]
    onehot = jnp.where(
        wid[:, None] == lax.broadcasted_iota(jnp.int32, (NB, VOCAB), 1),
        1.0, 0.0).astype(jnp.float32)
    h0 = jnp.dot(onehot, emb_ref[...], preferred_element_type=jnp.float32)
    h0_ref[...] = h0
    svecs = []
    for k in range(4):
        Wk = W_ref[k]
        zk = lax.dot_general(h0, Wk, (((1,), (1,)), ((), ())),
                             preferred_element_type=jnp.float32)
        z_ref[k] = zk
        a = A_ref[k, 0, :]
        s1 = jnp.dot(zk, a[:H], preferred_element_type=jnp.float32)
        s2 = jnp.dot(zk, a[H:], preferred_element_type=jnp.float32)
        svecs.append(s1)
        svecs.append(s2)
    sblk = jnp.stack(svecs, axis=0)
    s_ref[0] = sblk
    blkmax = jnp.max(sblk, axis=1, keepdims=True)

    @pl.when(i == 0)
    def _():
        bmax_ref[...] = jnp.full((8, 128), -1e30, jnp.float32)
    bmax_ref[...] = jnp.maximum(bmax_ref[...], jnp.broadcast_to(blkmax, (8, 128)))


def _prep(wid3, emb, gat_W, gat_A):
    grid = N_NODES // NB
    return pl.pallas_call(
        _prep_body,
        grid=(grid,),
        in_specs=[
            pl.BlockSpec((1, 1, NB), lambda i: (i, 0, 0)),
            pl.BlockSpec((VOCAB, H), lambda i: (0, 0)),
            pl.BlockSpec((4, H, H), lambda i: (0, 0, 0)),
            pl.BlockSpec((4, 1, 2 * H), lambda i: (0, 0, 0)),
        ],
        out_specs=[
            pl.BlockSpec((NB, H), lambda i: (i, 0)),
            pl.BlockSpec((4, NB, H), lambda i: (0, i, 0)),
            pl.BlockSpec((1, 8, NB), lambda i: (i, 0, 0)),
            pl.BlockSpec((8, 128), lambda i: (0, 0)),
        ],
        out_shape=[
            jax.ShapeDtypeStruct((N_NODES, H), jnp.float32),
            jax.ShapeDtypeStruct((4, N_NODES, H), jnp.float32),
            jax.ShapeDtypeStruct((N_NODES // NB, 8, NB), jnp.float32),
            jax.ShapeDtypeStruct((8, 128), jnp.float32),
        ],
    )(wid3, emb, gat_W, gat_A)


def _edges_sc_body(z_ref, s_ref, src_ref, dst_ref, hp_ref, ss_ref,
                   sidx8, didx8, s1v, s2v, wbuf, ssl, rin, hmsg,
                   gs0, gs1):
    cid = lax.axis_index("c")
    sid = lax.axis_index("s")
    wid = sid * 2 + cid
    vcnt = jnp.clip(N_EDGES - wid * EPW, 0, EPW)

    zv = jnp.zeros((16,), jnp.float32)
    lanes = lax.iota(jnp.int32, 16)

    def _gather(i, pp, half, sem):
        idx = sidx8.at[pp].at[pl.ds(half * 64, 64)]
        return pltpu.async_copy(z_ref.at[i].at[idx],
                                rin.at[pl.ds(half * 64, 64)], sem)

    for i in range(4):
        def _zero_rin(k, _):
            for j in range(H // 16):
                rin[k, pl.ds(j * 16, 16)] = zv
            return 0
        lax.fori_loop(0, KB, _zero_rin, 0)
        for q in range(ROWS_T // KB):
            pltpu.sync_copy(rin,
                            hmsg.at[pl.ds(sid * ROWS_T + q * KB, KB)])

        def _zero_ssl(k, _):
            ssl[pl.ds(k * 16, 16)] = zv
            return 0
        lax.fori_loop(0, NPAD // 16, _zero_ssl, 0)
        pltpu.sync_copy(s_ref.at[2 * i], s1v)
        pltpu.sync_copy(s_ref.at[2 * i + 1], s2v)
        plsc.subcore_barrier()

        def _sbatch(sb, _):
            pltpu.sync_copy(src_ref.at[wid].at[pl.ds(sb * 8, 8)], sidx8)
            pltpu.sync_copy(dst_ref.at[wid].at[pl.ds(sb * 8, 8)], didx8)
            _gather(i, 0, 0, gs0)
            _gather(i, 0, 1, gs1)

            def _pair(pp, _2):
                b = sb * 8 + pp
                for j in range(KB // 16):
                    s16 = sidx8[pp, pl.ds(j * 16, 16)]
                    d16 = didx8[pp, pl.ds(j * 16, 16)]
                    g1 = plsc.load_gather(s1v, [s16])
                    g2 = plsc.load_gather(s2v, [d16])
                    e = g1 + g2
                    e = jnp.maximum(e, 0.01 * e)
                    w = jnp.exp(e)
                    eidx = b * KB + j * 16 + lanes
                    w = jnp.where(eidx < vcnt, w, 0.0)
                    wbuf[pl.ds(j * 16, 16)] = w
                    plsc.addupdate_scatter(ssl, [d16], w)

                def _scale(jj, _3):
                    w16 = wbuf[pl.ds(jj * 16, 16)]
                    for l in range(16):
                        s = w16[l]
                        k = jj * 16 + l
                        for j in range(H // 16):
                            rin[k, pl.ds(j * 16, 16)] = (
                                rin[k, pl.ds(j * 16, 16)] * s)
                    return 0
                pltpu.make_async_copy(
                    z_ref.at[i].at[sidx8.at[pp].at[pl.ds(0, 64)]],
                    rin.at[pl.ds(0, 64)], gs0).wait()
                lax.fori_loop(0, KB // 32, _scale, 0)
                pltpu.make_async_copy(
                    z_ref.at[i].at[sidx8.at[pp].at[pl.ds(64, 64)]],
                    rin.at[pl.ds(64, 64)], gs1).wait()
                lax.fori_loop(KB // 32, KB // 16, _scale, 0)
                pltpu.sync_copy(rin, hmsg.at[didx8.at[pp]], add=True)

                @pl.when(pp < 7)
                def _():
                    _gather(i, pp + 1, 0, gs0)
                    _gather(i, pp + 1, 1, gs1)
                return 0
            lax.fori_loop(0, 8, _pair, 0)
            return 0
        lax.fori_loop(0, SBN, _sbatch, 0)
        pltpu.sync_copy(ssl, ss_ref.at[i].at[cid].at[sid])
        plsc.subcore_barrier()
        pltpu.sync_copy(hmsg.at[pl.ds(sid * ROWS_T, ROWS_T)],
                        hp_ref.at[i].at[cid].at[pl.ds(sid * ROWS_T, ROWS_T)])
        plsc.subcore_barrier()


def _edges_sc(z_all, svec, srcw, dstw):
    mesh = plsc.VectorSubcoreMesh(core_axis_name="c", subcore_axis_name="s")
    f = pl.kernel(
        _edges_sc_body, mesh=mesh,
        compiler_params=pltpu.CompilerParams(needs_layout_passes=False),
        out_type=[
            jax.ShapeDtypeStruct((4, 2, NPAD, H), jnp.float32),
            jax.ShapeDtypeStruct((4, 2, 16, NPAD), jnp.float32),
        ],
        scratch_types=[
            pltpu.VMEM((8, KB), jnp.int32),
            pltpu.VMEM((8, KB), jnp.int32),
            pltpu.VMEM((N_NODES,), jnp.float32),
            pltpu.VMEM((N_NODES,), jnp.float32),
            pltpu.VMEM((KB,), jnp.float32),
            pltpu.VMEM((NPAD,), jnp.float32),
            pltpu.VMEM((KB, H), jnp.float32),
            pltpu.VMEM_SHARED((NPAD, H), jnp.float32),
            pltpu.SemaphoreType.DMA,
            pltpu.SemaphoreType.DMA,
        ],
    )
    return f(z_all, svec, srcw, dstw)


def _stats_body(hp_ref, ss_ref, hr_ref, st_ref):
    i = pl.program_id(0)

    @pl.when(i == 0)
    def _():
        st_ref[...] = jnp.zeros((8, 128), jnp.float32)
    rows = []
    for k in range(4):
        v = hp_ref[k, 0] + hp_ref[k, 1]
        s = jnp.sum(ss_ref[k], axis=(0, 1))
        hr = jnp.maximum(v / jnp.maximum(s, 1e-16)[:, None], 0.0)
        hr_ref[k] = hr
        rows.append(jnp.sum(hr, axis=0))
        rows.append(jnp.sum(hr * hr, axis=0))
    st_ref[...] = st_ref[...] + jnp.stack(rows, axis=0)


def _stats(hp, ssp):
    grid = NPAD // NBS
    return pl.pallas_call(
        _stats_body,
        grid=(grid,),
        in_specs=[
            pl.BlockSpec((4, 2, NBS, H), lambda i: (0, 0, i, 0)),
            pl.BlockSpec((4, 2, 16, NBS), lambda i: (0, 0, 0, i)),
        ],
        out_specs=[
            pl.BlockSpec((4, NBS, H), lambda i: (0, i, 0)),
            pl.BlockSpec((8, 128), lambda i: (0, 0)),
        ],
        out_shape=[
            jax.ShapeDtypeStruct((4, NPAD, H), jnp.float32),
            jax.ShapeDtypeStruct((8, 128), jnp.float32),
        ],
    )(hp, ssp)


def _final_body(hr_ref, st_ref, g_ref, b_ref, Wo_ref, bo_ref, out_ref):
    st = st_ref[...]
    acc = jnp.broadcast_to(bo_ref[0, :], (NBS, H))
    for k in range(4):
        mu = st[2 * k] / float(N_NODES)
        var = st[2 * k + 1] / float(N_NODES) - mu * mu
        inv = lax.rsqrt(var + 1e-5)
        hb = (hr_ref[k] - mu[None, :]) * (inv * g_ref[k])[None, :] + b_ref[k][None, :]
        Wk = Wo_ref[:, k * H:(k + 1) * H]
        acc = acc + lax.dot_general(hb, Wk, (((1,), (1,)), ((), ())),
                                    preferred_element_type=jnp.float32)
    out_ref[...] = acc


def _finalize(hr, st, gamma, beta, W_out, b_out2):
    grid = NPAD // NBS
    return pl.pallas_call(
        _final_body,
        grid=(grid,),
        in_specs=[
            pl.BlockSpec((4, NBS, H), lambda i: (0, i, 0)),
            pl.BlockSpec((8, 128), lambda i: (0, 0)),
            pl.BlockSpec((4, H), lambda i: (0, 0)),
            pl.BlockSpec((4, H), lambda i: (0, 0)),
            pl.BlockSpec((H, 4 * H), lambda i: (0, 0)),
            pl.BlockSpec((8, H), lambda i: (0, 0)),
        ],
        out_specs=pl.BlockSpec((NBS, H), lambda i: (i, 0)),
        out_shape=jax.ShapeDtypeStruct((NPAD, H), jnp.float32),
    )(hr, st, gamma, beta, W_out, b_out2)


def _segmean_body(gid_ref, h0_ref, nh_ref, sum_ref, cnt_ref):
    i = pl.program_id(0)

    @pl.when(i == 0)
    def _():
        sum_ref[...] = jnp.zeros((2, GP, H), jnp.float32)
        cnt_ref[...] = jnp.zeros((8, GP), jnp.float32)
    gid = gid_ref[0, 0, :]
    onehot = jnp.where(
        gid[:, None] == lax.broadcasted_iota(jnp.int32, (NB, GP), 1),
        1.0, 0.0).astype(jnp.float32)
    sum_ref[0] += lax.dot_general(onehot, h0_ref[...], (((0,), (0,)), ((), ())),
                                  preferred_element_type=jnp.float32)
    sum_ref[1] += lax.dot_general(onehot, nh_ref[...], (((0,), (0,)), ((), ())),
                                  preferred_element_type=jnp.float32)
    cnt = jnp.sum(onehot, axis=0)
    cnt_ref[...] += jnp.broadcast_to(cnt[None, :], (8, GP))


def _segmean(gid3, h0, new_h):
    grid = N_NODES // NB
    return pl.pallas_call(
        _segmean_body,
        grid=(grid,),
        in_specs=[
            pl.BlockSpec((1, 1, NB), lambda i: (i, 0, 0)),
            pl.BlockSpec((NB, H), lambda i: (i, 0)),
            pl.BlockSpec((NB, H), lambda i: (i, 0)),
        ],
        out_specs=[
            pl.BlockSpec((2, GP, H), lambda i: (0, 0, 0)),
            pl.BlockSpec((8, GP), lambda i: (0, 0)),
        ],
        out_shape=[
            jax.ShapeDtypeStruct((2, GP, H), jnp.float32),
            jax.ShapeDtypeStruct((8, GP), jnp.float32),
        ],
    )(gid3, h0, new_h)


def _gru_cell(x, h, Wih, Whh, bih, bhh):
    gi = lax.dot_general(x, Wih, (((1,), (1,)), ((), ())),
                         preferred_element_type=jnp.float32) + bih[None, :]
    gh = lax.dot_general(h, Whh, (((1,), (1,)), ((), ())),
                         preferred_element_type=jnp.float32) + bhh[None, :]
    ir, iz, inn = gi[:, :H], gi[:, H:2 * H], gi[:, 2 * H:]
    hr, hz, hn = gh[:, :H], gh[:, H:2 * H], gh[:, 2 * H:]
    r = jax.nn.sigmoid(ir + hr)
    z = jax.nn.sigmoid(iz + hz)
    n = jnp.tanh(inn + r * hn)
    return (1.0 - z) * n + z * h


def _gru_body(sum_ref, cnt_ref, Wih0_ref, Whh0_ref, bih0_ref, bhh0_ref,
              Wih1_ref, Whh1_ref, bih1_ref, bhh1_ref, out_ref):
    cnt = jnp.maximum(cnt_ref[0, :], 1.0)[:, None]
    x0 = sum_ref[0] / cnt
    x1 = sum_ref[1] / cnt
    zero = jnp.zeros((GP, H), jnp.float32)
    f0 = _gru_cell(x0, zero, Wih0_ref[0], Whh0_ref[0], bih0_ref[0, :], bhh0_ref[0, :])
    f1 = _gru_cell(x1, f0, Wih0_ref[0], Whh0_ref[0], bih0_ref[0, :], bhh0_ref[0, :])
    b1 = _gru_cell(x1, zero, Wih0_ref[1], Whh0_ref[1], bih0_ref[1, :], bhh0_ref[1, :])
    b0 = _gru_cell(x0, b1, Wih0_ref[1], Whh0_ref[1], bih0_ref[1, :], bhh0_ref[1, :])
    y0 = jnp.concatenate([f0, b0], axis=1)
    y1 = jnp.concatenate([f1, b1], axis=1)
    g0 = _gru_cell(y0, zero, Wih1_ref[0], Whh1_ref[0], bih1_ref[0, :], bhh1_ref[0, :])
    g1 = _gru_cell(y1, g0, Wih1_ref[0], Whh1_ref[0], bih1_ref[0, :], bhh1_ref[0, :])
    c1 = _gru_cell(y1, zero, Wih1_ref[1], Whh1_ref[1], bih1_ref[1, :], bhh1_ref[1, :])
    c0 = _gru_cell(y0, c1, Wih1_ref[1], Whh1_ref[1], bih1_ref[1, :], bhh1_ref[1, :])
    out_ref[...] = f1 + b0 + g1 + c0


def _gru(sums, cnts, Wih0, Whh0, bih0, bhh0, Wih1, Whh1, bih1, bhh1):
    full = lambda shape: pl.BlockSpec(shape, lambda: tuple(0 for _ in shape))
    return pl.pallas_call(
        _gru_body,
        in_specs=[
            full((2, GP, H)), full((8, GP)),
            full((2, 3 * H, H)), full((2, 3 * H, H)),
            full((2, 3 * H)), full((2, 3 * H)),
            full((2, 3 * H, 2 * H)), full((2, 3 * H, H)),
            full((2, 3 * H)), full((2, 3 * H)),
        ],
        out_specs=full((GP, H)),
        out_shape=jax.ShapeDtypeStruct((GP, H), jnp.float32),
    )(sums, cnts, Wih0, Whh0, bih0, bhh0, Wih1, Whh1, bih1, bhh1)


def kernel(emb, gat_W, gat_A, gat_gamma, gat_beta, W_out, b_out,
           gru_Wih0, gru_Whh0, gru_bih0, gru_bhh0,
           gru_Wih1, gru_Whh1, gru_bih1, gru_bhh1,
           wid, edge_index, graph_ids):
    wid3 = wid.astype(jnp.int32).reshape(N_NODES // NB, 1, NB)
    gid3 = graph_ids.astype(jnp.int32).reshape(N_NODES // NB, 1, NB)
    src = edge_index[0].astype(jnp.int32)
    dst = edge_index[1].astype(jnp.int32)

    h0, z_all, svec3, bmax = _prep(wid3, emb, gat_W, gat_A)
    svec = svec3.transpose(1, 0, 2).reshape(8, N_NODES)
    pad = NWORK * EPW - N_EDGES
    srcw = jnp.pad(src, (0, pad)).reshape(NWORK, NBATCH, KB)
    dstw = jnp.pad(dst, (0, pad)).reshape(NWORK, NBATCH, KB)
    hp, ssp = _edges_sc(z_all, svec, srcw, dstw)

    hr, st = _stats(hp, ssp)
    b_out2 = jnp.broadcast_to(b_out[None, :], (8, H))
    new_h = _finalize(hr, st, gat_gamma, gat_beta, W_out, b_out2)[:N_NODES]

    sums, cnts = _segmean(gid3, h0, new_h)
    hsum = _gru(sums, cnts, gru_Wih0, gru_Whh0, gru_bih0, gru_bhh0,
                gru_Wih1, gru_Whh1, gru_bih1, gru_bhh1)
    return new_h, hsum[:N_GRAPHS]


# speedup vs baseline: 11.0698x; 1.0082x over previous
import functools
import jax
import jax.numpy as jnp
from jax import lax
from jax.experimental import pallas as pl
from jax.experimental.pallas import tpu as pltpu
from jax.experimental.pallas import tpu_sc as plsc

N_NODES = 10000
N_GRAPHS = 500
VOCAB = 1000
H = 128
NB = 1000
GP = 512

N_EDGES = 320000
NWORK = 32
EPW = 10240
KB = 128
NBATCH = EPW // KB
SBN = NBATCH // 8
NPAD = 10240
ROWS_T = NPAD // 16
NBS = 1024


def _prep_body(wid_ref, emb_ref, W_ref, A_ref, h0_ref, z_ref, s_ref, bmax_ref):
    i = pl.program_id(0)
    wid = wid_ref[0, 0, :]
    onehot = jnp.where(
        wid[:, None] == lax.broadcasted_iota(jnp.int32, (NB, VOCAB), 1),
        1.0, 0.0).astype(jnp.float32)
    h0 = jnp.dot(onehot, emb_ref[...], preferred_element_type=jnp.float32)
    h0_ref[...] = h0
    svecs = []
    for k in range(4):
        Wk = W_ref[k]
        zk = lax.dot_general(h0, Wk, (((1,), (1,)), ((), ())),
                             preferred_element_type=jnp.float32)
        z_ref[k] = zk
        a = A_ref[k, 0, :]
        s1 = jnp.dot(zk, a[:H], preferred_element_type=jnp.float32)
        s2 = jnp.dot(zk, a[H:], preferred_element_type=jnp.float32)
        svecs.append(s1)
        svecs.append(s2)
    sblk = jnp.stack(svecs, axis=0)
    s_ref[0] = sblk
    blkmax = jnp.max(sblk, axis=1, keepdims=True)

    @pl.when(i == 0)
    def _():
        bmax_ref[...] = jnp.full((8, 128), -1e30, jnp.float32)
    bmax_ref[...] = jnp.maximum(bmax_ref[...], jnp.broadcast_to(blkmax, (8, 128)))


def _prep(wid3, emb, gat_W, gat_A):
    grid = N_NODES // NB
    return pl.pallas_call(
        _prep_body,
        grid=(grid,),
        in_specs=[
            pl.BlockSpec((1, 1, NB), lambda i: (i, 0, 0)),
            pl.BlockSpec((VOCAB, H), lambda i: (0, 0)),
            pl.BlockSpec((4, H, H), lambda i: (0, 0, 0)),
            pl.BlockSpec((4, 1, 2 * H), lambda i: (0, 0, 0)),
        ],
        out_specs=[
            pl.BlockSpec((NB, H), lambda i: (i, 0)),
            pl.BlockSpec((4, NB, H), lambda i: (0, i, 0)),
            pl.BlockSpec((1, 8, NB), lambda i: (i, 0, 0)),
            pl.BlockSpec((8, 128), lambda i: (0, 0)),
        ],
        out_shape=[
            jax.ShapeDtypeStruct((N_NODES, H), jnp.float32),
            jax.ShapeDtypeStruct((4, N_NODES, H), jnp.float32),
            jax.ShapeDtypeStruct((N_NODES // NB, 8, NB), jnp.float32),
            jax.ShapeDtypeStruct((8, 128), jnp.float32),
        ],
    )(wid3, emb, gat_W, gat_A)


def _edges_sc_body(z_ref, s_ref, src_ref, dst_ref, hp_ref, ss_ref,
                   sidx8, didx8, s1v, s2v, wbuf, ssl, rin, hmsg,
                   gs0, gs1):
    cid = lax.axis_index("c")
    sid = lax.axis_index("s")
    wid = sid * 2 + cid
    vcnt = jnp.clip(N_EDGES - wid * EPW, 0, EPW)

    zv = jnp.zeros((16,), jnp.float32)
    lanes = lax.iota(jnp.int32, 16)

    def _gather(i, pp, half, sem):
        idx = sidx8.at[pp].at[pl.ds(half * 64, 64)]
        return pltpu.async_copy(z_ref.at[i].at[idx],
                                rin.at[pl.ds(half * 64, 64)], sem)

    for i in range(4):
        @plsc.parallel_loop(0, KB, unroll=2)
        def _zero_rin(k):
            for j in range(H // 16):
                rin[k, pl.ds(j * 16, 16)] = zv
        for q in range(ROWS_T // KB):
            pltpu.sync_copy(rin,
                            hmsg.at[pl.ds(sid * ROWS_T + q * KB, KB)])

        @plsc.parallel_loop(0, NPAD // 16, unroll=4)
        def _zero_ssl(k):
            ssl[pl.ds(k * 16, 16)] = zv
        pltpu.sync_copy(s_ref.at[2 * i], s1v)
        pltpu.sync_copy(s_ref.at[2 * i + 1], s2v)
        plsc.subcore_barrier()

        def _sbatch(sb, _):
            pltpu.sync_copy(src_ref.at[wid].at[pl.ds(sb * 8, 8)], sidx8)
            pltpu.sync_copy(dst_ref.at[wid].at[pl.ds(sb * 8, 8)], didx8)
            _gather(i, 0, 0, gs0)
            _gather(i, 0, 1, gs1)

            def _pair(pp, _2):
                b = sb * 8 + pp
                for j in range(KB // 16):
                    s16 = sidx8[pp, pl.ds(j * 16, 16)]
                    d16 = didx8[pp, pl.ds(j * 16, 16)]
                    g1 = plsc.load_gather(s1v, [s16])
                    g2 = plsc.load_gather(s2v, [d16])
                    e = g1 + g2
                    e = jnp.maximum(e, 0.01 * e)
                    w = jnp.exp(e)
                    eidx = b * KB + j * 16 + lanes
                    w = jnp.where(eidx < vcnt, w, 0.0)
                    wbuf[pl.ds(j * 16, 16)] = w
                    plsc.addupdate_scatter(ssl, [d16], w)

                def _scale_body(jj):
                    w16 = wbuf[pl.ds(jj * 16, 16)]
                    for l in range(16):
                        s = w16[l]
                        k = jj * 16 + l
                        for j in range(H // 16):
                            rin[k, pl.ds(j * 16, 16)] = (
                                rin[k, pl.ds(j * 16, 16)] * s)
                pltpu.make_async_copy(
                    z_ref.at[i].at[sidx8.at[pp].at[pl.ds(0, 64)]],
                    rin.at[pl.ds(0, 64)], gs0).wait()
                plsc.parallel_loop(0, KB // 32)(_scale_body)
                pltpu.make_async_copy(
                    z_ref.at[i].at[sidx8.at[pp].at[pl.ds(64, 64)]],
                    rin.at[pl.ds(64, 64)], gs1).wait()
                plsc.parallel_loop(KB // 32, KB // 16)(_scale_body)
                pltpu.sync_copy(rin, hmsg.at[didx8.at[pp]], add=True)

                @pl.when(pp < 7)
                def _():
                    _gather(i, pp + 1, 0, gs0)
                    _gather(i, pp + 1, 1, gs1)
                return 0
            lax.fori_loop(0, 8, _pair, 0)
            return 0
        lax.fori_loop(0, SBN, _sbatch, 0)
        pltpu.sync_copy(ssl, ss_ref.at[i].at[cid].at[sid])
        plsc.subcore_barrier()
        pltpu.sync_copy(hmsg.at[pl.ds(sid * ROWS_T, ROWS_T)],
                        hp_ref.at[i].at[cid].at[pl.ds(sid * ROWS_T, ROWS_T)])
        plsc.subcore_barrier()


def _edges_sc(z_all, svec, srcw, dstw):
    mesh = plsc.VectorSubcoreMesh(core_axis_name="c", subcore_axis_name="s")
    f = pl.kernel(
        _edges_sc_body, mesh=mesh,
        compiler_params=pltpu.CompilerParams(needs_layout_passes=False),
        out_type=[
            jax.ShapeDtypeStruct((4, 2, NPAD, H), jnp.float32),
            jax.ShapeDtypeStruct((4, 2, 16, NPAD), jnp.float32),
        ],
        scratch_types=[
            pltpu.VMEM((8, KB), jnp.int32),
            pltpu.VMEM((8, KB), jnp.int32),
            pltpu.VMEM((N_NODES,), jnp.float32),
            pltpu.VMEM((N_NODES,), jnp.float32),
            pltpu.VMEM((KB,), jnp.float32),
            pltpu.VMEM((NPAD,), jnp.float32),
            pltpu.VMEM((KB, H), jnp.float32),
            pltpu.VMEM_SHARED((NPAD, H), jnp.float32),
            pltpu.SemaphoreType.DMA,
            pltpu.SemaphoreType.DMA,
        ],
    )
    return f(z_all, svec, srcw, dstw)


def _stats_body(hp_ref, ss_ref, hr_ref, st_ref):
    i = pl.program_id(0)

    @pl.when(i == 0)
    def _():
        st_ref[...] = jnp.zeros((8, 128), jnp.float32)
    rows = []
    for k in range(4):
        v = hp_ref[k, 0] + hp_ref[k, 1]
        s = jnp.sum(ss_ref[k], axis=(0, 1))
        hr = jnp.maximum(v / jnp.maximum(s, 1e-16)[:, None], 0.0)
        hr_ref[k] = hr
        rows.append(jnp.sum(hr, axis=0))
        rows.append(jnp.sum(hr * hr, axis=0))
    st_ref[...] = st_ref[...] + jnp.stack(rows, axis=0)


def _stats(hp, ssp):
    grid = NPAD // NBS
    return pl.pallas_call(
        _stats_body,
        grid=(grid,),
        in_specs=[
            pl.BlockSpec((4, 2, NBS, H), lambda i: (0, 0, i, 0)),
            pl.BlockSpec((4, 2, 16, NBS), lambda i: (0, 0, 0, i)),
        ],
        out_specs=[
            pl.BlockSpec((4, NBS, H), lambda i: (0, i, 0)),
            pl.BlockSpec((8, 128), lambda i: (0, 0)),
        ],
        out_shape=[
            jax.ShapeDtypeStruct((4, NPAD, H), jnp.float32),
            jax.ShapeDtypeStruct((8, 128), jnp.float32),
        ],
    )(hp, ssp)


def _final_body(hr_ref, st_ref, g_ref, b_ref, Wo_ref, bo_ref, out_ref):
    st = st_ref[...]
    acc = jnp.broadcast_to(bo_ref[0, :], (NBS, H))
    for k in range(4):
        mu = st[2 * k] / float(N_NODES)
        var = st[2 * k + 1] / float(N_NODES) - mu * mu
        inv = lax.rsqrt(var + 1e-5)
        hb = (hr_ref[k] - mu[None, :]) * (inv * g_ref[k])[None, :] + b_ref[k][None, :]
        Wk = Wo_ref[:, k * H:(k + 1) * H]
        acc = acc + lax.dot_general(hb, Wk, (((1,), (1,)), ((), ())),
                                    preferred_element_type=jnp.float32)
    out_ref[...] = acc


def _finalize(hr, st, gamma, beta, W_out, b_out2):
    grid = NPAD // NBS
    return pl.pallas_call(
        _final_body,
        grid=(grid,),
        in_specs=[
            pl.BlockSpec((4, NBS, H), lambda i: (0, i, 0)),
            pl.BlockSpec((8, 128), lambda i: (0, 0)),
            pl.BlockSpec((4, H), lambda i: (0, 0)),
            pl.BlockSpec((4, H), lambda i: (0, 0)),
            pl.BlockSpec((H, 4 * H), lambda i: (0, 0)),
            pl.BlockSpec((8, H), lambda i: (0, 0)),
        ],
        out_specs=pl.BlockSpec((NBS, H), lambda i: (i, 0)),
        out_shape=jax.ShapeDtypeStruct((NPAD, H), jnp.float32),
    )(hr, st, gamma, beta, W_out, b_out2)


def _segmean_body(gid_ref, h0_ref, nh_ref, sum_ref, cnt_ref):
    i = pl.program_id(0)

    @pl.when(i == 0)
    def _():
        sum_ref[...] = jnp.zeros((2, GP, H), jnp.float32)
        cnt_ref[...] = jnp.zeros((8, GP), jnp.float32)
    gid = gid_ref[0, 0, :]
    onehot = jnp.where(
        gid[:, None] == lax.broadcasted_iota(jnp.int32, (NB, GP), 1),
        1.0, 0.0).astype(jnp.float32)
    sum_ref[0] += lax.dot_general(onehot, h0_ref[...], (((0,), (0,)), ((), ())),
                                  preferred_element_type=jnp.float32)
    sum_ref[1] += lax.dot_general(onehot, nh_ref[...], (((0,), (0,)), ((), ())),
                                  preferred_element_type=jnp.float32)
    cnt = jnp.sum(onehot, axis=0)
    cnt_ref[...] += jnp.broadcast_to(cnt[None, :], (8, GP))


def _segmean(gid3, h0, new_h):
    grid = N_NODES // NB
    return pl.pallas_call(
        _segmean_body,
        grid=(grid,),
        in_specs=[
            pl.BlockSpec((1, 1, NB), lambda i: (i, 0, 0)),
            pl.BlockSpec((NB, H), lambda i: (i, 0)),
            pl.BlockSpec((NB, H), lambda i: (i, 0)),
        ],
        out_specs=[
            pl.BlockSpec((2, GP, H), lambda i: (0, 0, 0)),
            pl.BlockSpec((8, GP), lambda i: (0, 0)),
        ],
        out_shape=[
            jax.ShapeDtypeStruct((2, GP, H), jnp.float32),
            jax.ShapeDtypeStruct((8, GP), jnp.float32),
        ],
    )(gid3, h0, new_h)


def _gru_cell(x, h, Wih, Whh, bih, bhh):
    gi = lax.dot_general(x, Wih, (((1,), (1,)), ((), ())),
                         preferred_element_type=jnp.float32) + bih[None, :]
    gh = lax.dot_general(h, Whh, (((1,), (1,)), ((), ())),
                         preferred_element_type=jnp.float32) + bhh[None, :]
    ir, iz, inn = gi[:, :H], gi[:, H:2 * H], gi[:, 2 * H:]
    hr, hz, hn = gh[:, :H], gh[:, H:2 * H], gh[:, 2 * H:]
    r = jax.nn.sigmoid(ir + hr)
    z = jax.nn.sigmoid(iz + hz)
    n = jnp.tanh(inn + r * hn)
    return (1.0 - z) * n + z * h


def _gru_body(sum_ref, cnt_ref, Wih0_ref, Whh0_ref, bih0_ref, bhh0_ref,
              Wih1_ref, Whh1_ref, bih1_ref, bhh1_ref, out_ref):
    cnt = jnp.maximum(cnt_ref[0, :], 1.0)[:, None]
    x0 = sum_ref[0] / cnt
    x1 = sum_ref[1] / cnt
    zero = jnp.zeros((GP, H), jnp.float32)
    f0 = _gru_cell(x0, zero, Wih0_ref[0], Whh0_ref[0], bih0_ref[0, :], bhh0_ref[0, :])
    f1 = _gru_cell(x1, f0, Wih0_ref[0], Whh0_ref[0], bih0_ref[0, :], bhh0_ref[0, :])
    b1 = _gru_cell(x1, zero, Wih0_ref[1], Whh0_ref[1], bih0_ref[1, :], bhh0_ref[1, :])
    b0 = _gru_cell(x0, b1, Wih0_ref[1], Whh0_ref[1], bih0_ref[1, :], bhh0_ref[1, :])
    y0 = jnp.concatenate([f0, b0], axis=1)
    y1 = jnp.concatenate([f1, b1], axis=1)
    g0 = _gru_cell(y0, zero, Wih1_ref[0], Whh1_ref[0], bih1_ref[0, :], bhh1_ref[0, :])
    g1 = _gru_cell(y1, g0, Wih1_ref[0], Whh1_ref[0], bih1_ref[0, :], bhh1_ref[0, :])
    c1 = _gru_cell(y1, zero, Wih1_ref[1], Whh1_ref[1], bih1_ref[1, :], bhh1_ref[1, :])
    c0 = _gru_cell(y0, c1, Wih1_ref[1], Whh1_ref[1], bih1_ref[1, :], bhh1_ref[1, :])
    out_ref[...] = f1 + b0 + g1 + c0


def _gru(sums, cnts, Wih0, Whh0, bih0, bhh0, Wih1, Whh1, bih1, bhh1):
    full = lambda shape: pl.BlockSpec(shape, lambda: tuple(0 for _ in shape))
    return pl.pallas_call(
        _gru_body,
        in_specs=[
            full((2, GP, H)), full((8, GP)),
            full((2, 3 * H, H)), full((2, 3 * H, H)),
            full((2, 3 * H)), full((2, 3 * H)),
            full((2, 3 * H, 2 * H)), full((2, 3 * H, H)),
            full((2, 3 * H)), full((2, 3 * H)),
        ],
        out_specs=full((GP, H)),
        out_shape=jax.ShapeDtypeStruct((GP, H), jnp.float32),
    )(sums, cnts, Wih0, Whh0, bih0, bhh0, Wih1, Whh1, bih1, bhh1)


def kernel(emb, gat_W, gat_A, gat_gamma, gat_beta, W_out, b_out,
           gru_Wih0, gru_Whh0, gru_bih0, gru_bhh0,
           gru_Wih1, gru_Whh1, gru_bih1, gru_bhh1,
           wid, edge_index, graph_ids):
    wid3 = wid.astype(jnp.int32).reshape(N_NODES // NB, 1, NB)
    gid3 = graph_ids.astype(jnp.int32).reshape(N_NODES // NB, 1, NB)
    src = edge_index[0].astype(jnp.int32)
    dst = edge_index[1].astype(jnp.int32)

    h0, z_all, svec3, bmax = _prep(wid3, emb, gat_W, gat_A)
    svec = svec3.transpose(1, 0, 2).reshape(8, N_NODES)
    pad = NWORK * EPW - N_EDGES
    srcw = jnp.pad(src, (0, pad)).reshape(NWORK, NBATCH, KB)
    dstw = jnp.pad(dst, (0, pad)).reshape(NWORK, NBATCH, KB)
    hp, ssp = _edges_sc(z_all, svec, srcw, dstw)

    hr, st = _stats(hp, ssp)
    b_out2 = jnp.broadcast_to(b_out[None, :], (8, H))
    new_h = _finalize(hr, st, gat_gamma, gat_beta, W_out, b_out2)[:N_NODES]

    sums, cnts = _segmean(gid3, h0, new_h)
    hsum = _gru(sums, cnts, gru_Wih0, gru_Whh0, gru_bih0, gru_bhh0,
                gru_Wih1, gru_Whh1, gru_bih1, gru_bhh1)
    return new_h, hsum[:N_GRAPHS]
